# baseline (speedup 1.0000x reference)
import jax
import jax.numpy as jnp
from jax import lax
from jax.experimental import pallas as pl
from jax.experimental.pallas import tpu as pltpu
from jax.experimental.pallas import tpu_sc as plsc

B = 4096
T = 200
BT = B * T
H = 64

NC = 2
NS = 16
NW = NC * NS
CHUNK = 256
PER_WS = B // NW


def _sc_mesh():
  return plsc.VectorSubcoreMesh(core_axis_name="c", subcore_axis_name="s")


def _gather_stream(idx_hbm, tab, out_hbm, col, total, chunk, wid,
                   idx_v, rows_v, gsem, ssem):
  per_w = total // NW
  nch = per_w // chunk
  base0 = wid * per_w
  idx = idx_v.at[pl.ds(0, chunk)]
  rows = rows_v.at[pl.ds(0, chunk)]

  @pl.loop(0, nch)
  def _(i):
    base = base0 + i * chunk
    pltpu.sync_copy(idx_hbm.at[pl.ds(base, chunk)], idx)
    pltpu.async_copy(tab.at[idx], rows, gsem).wait()
    pltpu.async_copy(
        rows, out_hbm.at[pl.ds(base, chunk), pl.ds(col, 2 * H)], ssem).wait()


def _sc_ks_body(ik0, ik1, is0, is1, is2, kt0, kt1, st0, st1, st2,
                kn2, gk1, s2, gs1, idx_v, rows_v, gsem, ssem):
  wid = lax.axis_index("s") * NC + lax.axis_index("c")
  _gather_stream(ik0, kt0, kn2, 0, BT, CHUNK, wid, idx_v, rows_v, gsem, ssem)
  _gather_stream(ik1, kt1, gk1, 0, BT, CHUNK, wid, idx_v, rows_v, gsem, ssem)
  _gather_stream(is0, st0, s2, 0, B, PER_WS, wid, idx_v, rows_v, gsem, ssem)
  _gather_stream(is1, st1, gs1, 0, B, PER_WS, wid, idx_v, rows_v, gsem, ssem)
  _gather_stream(is2, st2, s2, 2 * H, B, PER_WS, wid, idx_v, rows_v,
                 gsem, ssem)


def _sc_o_body(io, ot, ob2, idx_v, rows_v, gsem, ssem):
  wid = lax.axis_index("s") * NC + lax.axis_index("c")
  _gather_stream(io, ot, ob2, 0, BT, CHUNK, wid, idx_v, rows_v, gsem, ssem)


def _sc_scratch():
  return [
      pltpu.VMEM((CHUNK,), jnp.int32),
      pltpu.VMEM((CHUNK, 2 * H), jnp.float32),
      pltpu.SemaphoreType.DMA,
      pltpu.SemaphoreType.DMA,
  ]


def _pair_consts(emb, bias, lo, hi):
  e = emb[lo:hi]
  bb = bias[lo:hi]
  n = e.shape[0]
  z = jnp.zeros((n // 2, H), e.dtype)
  el = jnp.concatenate([e[0::2], z], axis=1)[:, None, :]
  er = jnp.concatenate([z, e[1::2]], axis=1)[:, None, :]
  b2 = jnp.concatenate([bb[0::2], bb[1::2]], axis=1)[:, None, :]
  return el, er, b2


def _right_half(emb, bias, j):
  z = jnp.zeros((1, H), emb.dtype)
  er = jnp.concatenate([z, emb[j:j + 1]], axis=1)
  br = jnp.concatenate([z, bias[j:j + 1]], axis=1)
  return er, br


def kernel(s_cat, s_cont, k_cat, k_cont, o_cat, o_cont, target,
           s_cat_tables, k_cat_tables, o_cat_tables,
           s_cont_emb, s_cont_bias, k_cont_emb, k_cont_bias,
           o_cont_emb, o_cont_bias, tgt_emb, tgt_bias):
  f32 = jnp.float32
  ik0 = k_cat[:, :, 0].reshape(BT)
  ik1 = k_cat[:, :, 1].reshape(BT)
  io = o_cat[:, :, 0].reshape(BT)
  is0 = s_cat[:, 0, 0]
  is1 = s_cat[:, 0, 1]
  is2 = s_cat[:, 0, 2]
  kct = k_cont.reshape(BT, 4, 2).transpose(1, 0, 2)
  oct_ = o_cont.reshape(BT, 8, 1).transpose(1, 0, 2)
  sct = s_cont.reshape(B, 4, 1).transpose(1, 0, 2)
  tg = target.reshape(BT, 1)

  padr = lambda t: jnp.pad(t, ((0, 0), (0, H)))
  padl = lambda t: jnp.pad(t, ((0, 0), (H, 0)))
  kt0, kt1 = padr(k_cat_tables[0]), padl(k_cat_tables[1])
  ot = padr(o_cat_tables[0])
  st0, st1, st2 = padr(s_cat_tables[0]), padl(s_cat_tables[1]), padr(
      s_cat_tables[2])

  sc_ks = pl.kernel(
      _sc_ks_body,
      out_type=(
          jax.ShapeDtypeStruct((BT, 10 * H), f32),
          jax.ShapeDtypeStruct((BT, 2 * H), f32),
          jax.ShapeDtypeStruct((B, 7 * H), f32),
          jax.ShapeDtypeStruct((B, 2 * H), f32),
      ),
      mesh=_sc_mesh(),
      scratch_types=_sc_scratch(),
  )
  kn2, gk1, s2, gs1 = sc_ks(ik0, ik1, is0, is1, is2, kt0, kt1, st0, st1, st2)

  sc_o = pl.kernel(
      _sc_o_body,
      out_type=jax.ShapeDtypeStruct((BT, 9 * H), f32),
      mesh=_sc_mesh(),
      scratch_types=_sc_scratch(),
  )
  ob2 = sc_o(io, ot)

  CB = 2048

  kel, ker, kb2 = _pair_consts(k_cont_emb, k_cont_bias, 0, 8)

  def kn_body(in0_ref, g1_ref, x_ref, el_ref, er_ref, b2_ref, out_ref):
    out_ref[:, 0:2 * H] = in0_ref[...] + g1_ref[...]
    for p in range(4):
      x = x_ref[p]
      out_ref[:, (2 + 2 * p) * H:(4 + 2 * p) * H] = (
          x[:, 0:1] * el_ref[p] + x[:, 1:2] * er_ref[p] + b2_ref[p])

  kn2 = pl.pallas_call(
      kn_body,
      grid=(BT // CB,),
      in_specs=[
          pl.BlockSpec((CB, 2 * H), lambda i: (i, 0)),
          pl.BlockSpec((CB, 2 * H), lambda i: (i, 0)),
          pl.BlockSpec((4, CB, 2), lambda i: (0, i, 0)),
          pl.BlockSpec((4, 1, 2 * H), lambda i: (0, 0, 0)),
          pl.BlockSpec((4, 1, 2 * H), lambda i: (0, 0, 0)),
          pl.BlockSpec((4, 1, 2 * H), lambda i: (0, 0, 0)),
      ],
      out_specs=pl.BlockSpec((CB, 10 * H), lambda i: (i, 0)),
      out_shape=jax.ShapeDtypeStruct((BT, 10 * H), f32),
      input_output_aliases={0: 0},
  )(kn2, gk1, kct, kel, ker, kb2)

  oel, oer, ob23 = _pair_consts(o_cont_emb, o_cont_bias, 1, 7)
  oer0, obr0 = _right_half(o_cont_emb, o_cont_bias, 0)
  oe7 = o_cont_emb[7:8]
  ob7 = o_cont_bias[7:8]

  def obs_body(in0_ref, x_ref, el_ref, er_ref, b2_ref, er0_ref, br0_ref,
               e7_ref, b7_ref, out_ref):
    out_ref[:, 0:2 * H] = (in0_ref[...] + x_ref[0] * er0_ref[...]
                           + br0_ref[...])
    for p in range(3):
      xl = x_ref[1 + 2 * p]
      xr = x_ref[2 + 2 * p]
      out_ref[:, (2 + 2 * p) * H:(4 + 2 * p) * H] = (
          xl * el_ref[p] + xr * er_ref[p] + b2_ref[p])
    out_ref[:, 8 * H:9 * H] = x_ref[7] * e7_ref[...] + b7_ref[...]

  ob2 = pl.pallas_call(
      obs_body,
      grid=(BT // CB,),
      in_specs=[
          pl.BlockSpec((CB, 2 * H), lambda i: (i, 0)),
          pl.BlockSpec((8, CB, 1), lambda i: (0, i, 0)),
          pl.BlockSpec((3, 1, 2 * H), lambda i: (0, 0, 0)),
          pl.BlockSpec((3, 1, 2 * H), lambda i: (0, 0, 0)),
          pl.BlockSpec((3, 1, 2 * H), lambda i: (0, 0, 0)),
          pl.BlockSpec((1, 2 * H), lambda i: (0, 0)),
          pl.BlockSpec((1, 2 * H), lambda i: (0, 0)),
          pl.BlockSpec((1, H), lambda i: (0, 0)),
          pl.BlockSpec((1, H), lambda i: (0, 0)),
      ],
      out_specs=pl.BlockSpec((CB, 9 * H), lambda i: (i, 0)),
      out_shape=jax.ShapeDtypeStruct((BT, 9 * H), f32),
      input_output_aliases={0: 0},
  )(ob2, oct_, oel, oer, ob23, oer0, obr0, oe7, ob7)

  sel12, ser12, sb12 = _pair_consts(s_cont_emb, s_cont_bias, 1, 3)
  ser0, sbr0 = _right_half(s_cont_emb, s_cont_bias, 0)
  se3 = s_cont_emb[3:4]
  sb3 = s_cont_bias[3:4]

  def s_body(in_ref, g1_ref, x_ref, el_ref, er_ref, b2_ref, er0_ref, br0_ref,
             e3_ref, b3_ref, out_ref):
    out_ref[:, 0:2 * H] = in_ref[:, 0:2 * H] + g1_ref[...]
    out_ref[:, 2 * H:4 * H] = (in_ref[:, 2 * H:4 * H]
                               + x_ref[0] * er0_ref[...] + br0_ref[...])
    out_ref[:, 4 * H:6 * H] = (x_ref[1] * el_ref[0] + x_ref[2] * er_ref[0]
                               + b2_ref[0])
    out_ref[:, 6 * H:7 * H] = x_ref[3] * e3_ref[...] + b3_ref[...]

  SB = 1024
  s2 = pl.pallas_call(
      s_body,
      grid=(B // SB,),
      in_specs=[
          pl.BlockSpec((SB, 4 * H), lambda i: (i, 0)),
          pl.BlockSpec((SB, 2 * H), lambda i: (i, 0)),
          pl.BlockSpec((4, SB, 1), lambda i: (0, i, 0)),
          pl.BlockSpec((1, 1, 2 * H), lambda i: (0, 0, 0)),
          pl.BlockSpec((1, 1, 2 * H), lambda i: (0, 0, 0)),
          pl.BlockSpec((1, 1, 2 * H), lambda i: (0, 0, 0)),
          pl.BlockSpec((1, 2 * H), lambda i: (0, 0)),
          pl.BlockSpec((1, 2 * H), lambda i: (0, 0)),
          pl.BlockSpec((1, H), lambda i: (0, 0)),
          pl.BlockSpec((1, H), lambda i: (0, 0)),
      ],
      out_specs=pl.BlockSpec((SB, 7 * H), lambda i: (i, 0)),
      out_shape=jax.ShapeDtypeStruct((B, 7 * H), f32),
      input_output_aliases={0: 0},
  )(s2, gs1, sct, sel12, ser12, sb12, ser0, sbr0, se3, sb3)

  def tgt_body(t_ref, e_ref, b_ref, out_ref):
    out_ref[...] = t_ref[...] * e_ref[...] + b_ref[...]

  tg2 = pl.pallas_call(
      tgt_body,
      grid=(BT // CB,),
      in_specs=[
          pl.BlockSpec((CB, 1), lambda i: (i, 0)),
          pl.BlockSpec((1, H), lambda i: (0, 0)),
          pl.BlockSpec((1, H), lambda i: (0, 0)),
      ],
      out_specs=pl.BlockSpec((CB, H), lambda i: (i, 0)),
      out_shape=jax.ShapeDtypeStruct((BT, H), f32),
  )(tg, tgt_emb, tgt_bias)

  return (s2.reshape(B, 7, H),
          kn2.reshape(B, T, 10, H),
          ob2.reshape(B, T, 9, H),
          tg2.reshape(B, T, 1, H))
